# baseline (speedup 1.0000x reference)
import functools

import jax
import jax.numpy as jnp
from jax import lax
from jax.experimental import pallas as pl
from jax.experimental.pallas import tpu as pltpu
from jax.experimental.pallas import tpu_sc as plsc

_B = 16384
_F = 26
_V = 100000
_D = 32
_DENSE = 13
_IN = _DENSE + _F * _D
_H = 8

_NC = 2
_NS = 16
_NW = _NC * _NS
_ROWS = _B * _F
_BPW = _B // _NW
_CH = 128
_NCB = _BPW // _CH


@functools.cache
def _make_sc_gather():
    mesh = plsc.VectorSubcoreMesh(core_axis_name="c", subcore_axis_name="s")

    @functools.partial(
        pl.kernel,
        mesh=mesh,
        out_type=jax.ShapeDtypeStruct((_ROWS, _D), jnp.float32),
        scratch_types=[
            pltpu.VMEM((_F, _BPW), jnp.int32),
            pltpu.VMEM((_CH, _D), jnp.float32),
            pltpu.VMEM((_CH,), jnp.int32),
            pltpu.VMEM((_CH,), jnp.int32),
            pltpu.SemaphoreType.DMA,
            pltpu.SemaphoreType.DMA,
        ],
        compiler_params=pltpu.CompilerParams(use_tc_tiling_on_sc=False),
    )
    def _sc_gather(idxT_hbm, table_hbm, out_hbm, idx_v, rows_v, patt_v, oidx_v,
                   gsem, osem):
        wid = lax.axis_index("s") * _NC + lax.axis_index("c")
        b0 = wid * _BPW
        pltpu.sync_copy(idxT_hbm.at[:, pl.ds(b0, _BPW)], idx_v)
        for k in range(_CH // 16):
            lane = lax.broadcasted_iota(jnp.int32, (16,), 0)
            patt_v[pl.ds(k * 16, 16)] = (lane + 16 * k) * _F

        def chunk(t, carry):
            f = t // _NCB
            j = t % _NCB
            src = table_hbm.at[f].at[idx_v.at[f, pl.ds(j * _CH, _CH)]]
            pltpu.async_copy(src, rows_v, gsem).wait()
            base_o = (b0 + j * _CH) * _F + f
            for k in range(_CH // 16):
                oidx_v[pl.ds(k * 16, 16)] = patt_v[pl.ds(k * 16, 16)] + base_o
            pltpu.async_copy(rows_v, out_hbm.at[oidx_v], osem).wait()
            return carry

        lax.fori_loop(0, _F * _NCB, chunk, 0)

    return _sc_gather


def _mlp_body(dense_ref, emb_ref, w1_ref, b1_ref, w2_ref, b2_ref, out_ref):
    x = jnp.concatenate([dense_ref[...], emb_ref[...]], axis=1)
    h = jnp.dot(x, w1_ref[...], preferred_element_type=jnp.float32) + b1_ref[...]
    h = jnp.where(h >= 0, h, 0.02 * h)
    g = jnp.dot(h, w2_ref[...], preferred_element_type=jnp.float32) + b2_ref[...]
    out_ref[...] = x * jax.nn.sigmoid(g)


_BLK = 1024


@jax.jit
def kernel(dense, indices, tables, W1, b1, W2, b2):
    idxT = indices.astype(jnp.int32).T

    emb = _make_sc_gather()(idxT, tables)
    emb = emb.reshape(_B, _F * _D)

    grid = (_B // _BLK,)
    out = pl.pallas_call(
        _mlp_body,
        grid=grid,
        in_specs=[
            pl.BlockSpec((_BLK, _DENSE), lambda i: (i, 0)),
            pl.BlockSpec((_BLK, _F * _D), lambda i: (i, 0)),
            pl.BlockSpec((_IN, _H), lambda i: (0, 0)),
            pl.BlockSpec((1, _H), lambda i: (0, 0)),
            pl.BlockSpec((_H, _IN), lambda i: (0, 0)),
            pl.BlockSpec((1, _IN), lambda i: (0, 0)),
        ],
        out_specs=pl.BlockSpec((_BLK, _IN), lambda i: (i, 0)),
        out_shape=jax.ShapeDtypeStruct((_B, _IN), jnp.float32),
    )(dense, emb, W1, b1.reshape(1, _H), W2, b2.reshape(1, _IN))
    return out


# speedup vs baseline: 1.0477x; 1.0477x over previous
import functools

import jax
import jax.numpy as jnp
from jax import lax
from jax.experimental import pallas as pl
from jax.experimental.pallas import tpu as pltpu
from jax.experimental.pallas import tpu_sc as plsc

_B = 16384
_F = 26
_V = 100000
_D = 32
_DENSE = 13
_IN = _DENSE + _F * _D
_H = 8

_NC = 2
_NS = 16
_NW = _NC * _NS
_ROWS = _B * _F
_RPW = _ROWS // _NW
_CH = 128
_GRP = 8
_GROWS = _CH * _GRP
_NGRP = _RPW // _GROWS
_NCH = _RPW // _CH


@functools.cache
def _make_sc_gather():
    mesh = plsc.VectorSubcoreMesh(core_axis_name="c", subcore_axis_name="s")

    @functools.partial(
        pl.kernel,
        mesh=mesh,
        out_type=jax.ShapeDtypeStruct((_ROWS, _D), jnp.float32),
        compiler_params=pltpu.CompilerParams(use_tc_tiling_on_sc=False),
        scratch_types=[
            pltpu.VMEM((_NCH, _CH), jnp.int32),
            pltpu.VMEM((_GROWS, _D), jnp.float32),
            pltpu.SemaphoreType.DMA,
        ],
    )
    def _sc_gather(idx_hbm, tab_hbm, out_hbm, idx_v, rows_v, gsem):
        wid = lax.axis_index("s") * _NC + lax.axis_index("c")
        pltpu.sync_copy(idx_hbm.at[wid], idx_v)
        base = wid * _RPW

        def group(g, carry):
            cps = [
                pltpu.async_copy(
                    tab_hbm.at[idx_v.at[g * _GRP + j]],
                    rows_v.at[pl.ds(j * _CH, _CH)],
                    gsem,
                )
                for j in range(_GRP)
            ]
            for cp in cps:
                cp.wait()
            pltpu.sync_copy(rows_v, out_hbm.at[pl.ds(base + g * _GROWS, _GROWS)])
            return carry

        lax.fori_loop(0, _NGRP, group, 0)

    return _sc_gather


def _mlp_body(d_ref, e_ref, w1_ref, b1_ref, w2_ref, b2_ref, out_ref):
    x = jnp.concatenate([d_ref[...], e_ref[...]], axis=1)
    h = jnp.dot(x, w1_ref[...], preferred_element_type=jnp.float32) + b1_ref[...]
    h = jnp.where(h >= 0, h, 0.02 * h)
    g = jnp.dot(h, w2_ref[...], preferred_element_type=jnp.float32) + b2_ref[...]
    out_ref[...] = x * jax.nn.sigmoid(g)


_BLK = 1024


@jax.jit
def kernel(dense, indices, tables, W1, b1, W2, b2):
    idx = indices.astype(jnp.int32) + jnp.arange(_F, dtype=jnp.int32) * _V
    idx = idx.reshape(_NW, _NCH, _CH)
    tab = tables.reshape(_F * _V, _D)

    emb = _make_sc_gather()(idx, tab)
    emb = emb.reshape(_B, _F * _D)

    grid = (_B // _BLK,)
    out = pl.pallas_call(
        _mlp_body,
        grid=grid,
        in_specs=[
            pl.BlockSpec((_BLK, _DENSE), lambda i: (i, 0)),
            pl.BlockSpec((_BLK, _F * _D), lambda i: (i, 0)),
            pl.BlockSpec((_IN, _H), lambda i: (0, 0)),
            pl.BlockSpec((1, _H), lambda i: (0, 0)),
            pl.BlockSpec((_H, _IN), lambda i: (0, 0)),
            pl.BlockSpec((1, _IN), lambda i: (0, 0)),
        ],
        out_specs=pl.BlockSpec((_BLK, _IN), lambda i: (i, 0)),
        out_shape=jax.ShapeDtypeStruct((_B, _IN), jnp.float32),
    )(dense, emb, W1, b1.reshape(1, _H), W2, b2.reshape(1, _IN))
    return out


# speedup vs baseline: 3.0232x; 2.8856x over previous
import functools

import jax
import jax.numpy as jnp
from jax import lax
from jax.experimental import pallas as pl
from jax.experimental.pallas import tpu as pltpu
from jax.experimental.pallas import tpu_sc as plsc

_B = 16384
_F = 26
_V = 100000
_D = 32
_DENSE = 13
_IN = _DENSE + _F * _D
_H = 8

_NC = 2
_NS = 16
_NW = _NC * _NS
_PL = _F * _D
_PPW = _PL // _NW
_BC = 4096
_NBC = _B // _BC


@functools.cache
def _make_sc_gather():
    mesh = plsc.VectorSubcoreMesh(core_axis_name="c", subcore_axis_name="s")

    @functools.partial(
        pl.kernel,
        mesh=mesh,
        out_type=jax.ShapeDtypeStruct((_F, _D, _B), jnp.float32),
        compiler_params=pltpu.CompilerParams(needs_layout_passes=False),
        scratch_types=[
            pltpu.VMEM((_V,), jnp.float32),
            pltpu.VMEM((_B,), jnp.int32),
            pltpu.VMEM((_BC,), jnp.float32),
            pltpu.SemaphoreType.DMA,
            pltpu.SemaphoreType.DMA,
            pltpu.SemaphoreType.DMA,
        ],
    )
    def _sc_gather(idxT_hbm, tablesT_hbm, out_hbm, plane_v, idx_v, out_v,
                   psem, isem, osem):
        wid = lax.axis_index("s") * _NC + lax.axis_index("c")

        def plane_loop(p, carry):
            pid = wid * _PPW + p
            f = pid // _D
            d = pid % _D
            cp_p = pltpu.async_copy(tablesT_hbm.at[f, d], plane_v, psem)
            cp_i = pltpu.async_copy(idxT_hbm.at[f], idx_v, isem)
            cp_p.wait()
            cp_i.wait()

            def bchunk(c, carry2):
                def gblock(k, carry3):
                    iv = idx_v[pl.ds(c * _BC + k * 16, 16)]
                    out_v[pl.ds(k * 16, 16)] = plsc.load_gather(plane_v, [iv])
                    return carry3

                lax.fori_loop(0, _BC // 16, gblock, 0)
                pltpu.async_copy(
                    out_v, out_hbm.at[f, d, pl.ds(c * _BC, _BC)], osem
                ).wait()
                return carry2

            lax.fori_loop(0, _NBC, bchunk, 0)
            return carry

        lax.fori_loop(0, _PPW, plane_loop, 0)

    return _sc_gather


def _mlp_body(dT_ref, eT_ref, w1dT_ref, w1eT_ref, b1_ref, w2T_ref, b2_ref,
              outT_ref):
    dT = dT_ref[...]
    eT = eT_ref[...]
    hT = (jnp.dot(w1dT_ref[...], dT, preferred_element_type=jnp.float32)
          + jnp.dot(w1eT_ref[...], eT, preferred_element_type=jnp.float32)
          + b1_ref[...])
    hT = jnp.where(hT >= 0, hT, 0.02 * hT)
    gT = (jnp.dot(w2T_ref[...], hT, preferred_element_type=jnp.float32)
          + b2_ref[...])
    gT = jax.nn.sigmoid(gT)
    outT_ref[...] = jnp.concatenate([dT, eT], axis=0) * gT


_BLK = 1024


@jax.jit
def kernel(dense, indices, tables, W1, b1, W2, b2):
    idxT = indices.astype(jnp.int32).T
    tablesT = tables.transpose(0, 2, 1)
    denseT = dense.T

    embT = _make_sc_gather()(idxT, tablesT)
    embT2 = embT.reshape(_F * _D, _B)

    grid = (_B // _BLK,)
    outT = pl.pallas_call(
        _mlp_body,
        grid=grid,
        in_specs=[
            pl.BlockSpec((_DENSE, _BLK), lambda i: (0, i)),
            pl.BlockSpec((_F * _D, _BLK), lambda i: (0, i)),
            pl.BlockSpec((_H, _DENSE), lambda i: (0, 0)),
            pl.BlockSpec((_H, _F * _D), lambda i: (0, 0)),
            pl.BlockSpec((_H, 1), lambda i: (0, 0)),
            pl.BlockSpec((_IN, _H), lambda i: (0, 0)),
            pl.BlockSpec((_IN, 1), lambda i: (0, 0)),
        ],
        out_specs=pl.BlockSpec((_IN, _BLK), lambda i: (0, i)),
        out_shape=jax.ShapeDtypeStruct((_IN, _B), jnp.float32),
    )(denseT, embT2, W1[:_DENSE].T, W1[_DENSE:].T, b1.reshape(_H, 1),
      W2.T, b2.reshape(_IN, 1))
    return outT.T
